# baseline (speedup 1.0000x reference)
import functools

import jax
import jax.numpy as jnp
from jax import lax
from jax.experimental import pallas as pl
from jax.experimental.pallas import tpu as pltpu
from jax.experimental.pallas import tpu_sc as plsc

_NC = 2
_NS = 16
_NW = _NC * _NS
_L = 16
_SUB = 100
_SUBS_PER_CHUNK = 8
_CHUNK = _SUB * _SUBS_PER_CHUNK


def _emb_body(S, D, n_chunks,
              ids_hbm, tok_hbm, pos_hbm, out_hbm,
              idx_v, rows_v, pos_v, sem):
    wid = lax.axis_index("s") * _NC + lax.axis_index("c")
    pltpu.sync_copy(pos_hbm, pos_v)
    seqs_per_chunk = _CHUNK // S
    nvec = D // _L

    def chunk_body(c, carry):
        chunk_id = wid * n_chunks + c
        base = chunk_id * _CHUNK
        pltpu.sync_copy(
            ids_hbm.at[pl.ds(chunk_id * _SUBS_PER_CHUNK, _SUBS_PER_CHUNK)],
            idx_v)
        copies = [
            pltpu.async_copy(tok_hbm.at[idx_v.at[k]],
                             rows_v.at[pl.ds(k * _SUB, _SUB)], sem)
            for k in range(_SUBS_PER_CHUNK)
        ]
        for cp in copies:
            cp.wait()

        def pos_body(p, inner):
            pv = [pos_v[p, pl.ds(j * _L, _L)] for j in range(nvec)]
            for s_i in range(seqs_per_chunk):
                r = s_i * S + p
                for j in range(nvec):
                    rows_v[r, pl.ds(j * _L, _L)] = (
                        rows_v[r, pl.ds(j * _L, _L)] + pv[j])
            return inner

        lax.fori_loop(0, S, pos_body, 0)
        pltpu.sync_copy(rows_v, out_hbm.at[pl.ds(base, _CHUNK)])
        return carry

    lax.fori_loop(0, n_chunks, chunk_body, 0)


def kernel(input_ids, token_table, position_table):
    B, S = input_ids.shape
    V, D = token_table.shape
    N = B * S
    assert N % (_NW * _CHUNK) == 0 and _CHUNK % S == 0
    assert N % _SUB == 0 and D % _L == 0
    n_chunks = N // (_NW * _CHUNK)

    ids2 = input_ids.reshape(N // _SUB, _SUB).astype(jnp.int32)
    mesh = plsc.VectorSubcoreMesh(core_axis_name="c", subcore_axis_name="s",
                                  num_cores=_NC, num_subcores=_NS)
    k = pl.kernel(
        functools.partial(_emb_body, S, D, n_chunks),
        out_type=jax.ShapeDtypeStruct((N, D), jnp.float32),
        mesh=mesh,
        scratch_types=[
            pltpu.VMEM((_SUBS_PER_CHUNK, _SUB), jnp.int32),
            pltpu.VMEM((_CHUNK, D), jnp.float32),
            pltpu.VMEM((S, D), jnp.float32),
            pltpu.SemaphoreType.DMA,
        ],
        compiler_params=pltpu.CompilerParams(use_tc_tiling_on_sc=False),
    )
    out = k(ids2, token_table, position_table.astype(jnp.float32))
    return out.reshape(B, S, D)


# speedup vs baseline: 1.0772x; 1.0772x over previous
import functools

import jax
import jax.numpy as jnp
from jax import lax
from jax.experimental import pallas as pl
from jax.experimental.pallas import tpu as pltpu
from jax.experimental.pallas import tpu_sc as plsc

_NC = 2
_NS = 16
_NW = _NC * _NS
_L = 16
_SUB = 100
_SPC = 4
_CHUNK = _SUB * _SPC
_NBUF = 4


def _emb_body(S, D, n_chunks,
              ids_hbm, tok_hbm, pos_hbm, out_hbm,
              idx_v, rows_v, pos_v, *sems):
    gsem, osem = sems[:_NBUF], sems[_NBUF:]
    wid = lax.axis_index("s") * _NC + lax.axis_index("c")
    pltpu.sync_copy(pos_hbm, pos_v)
    seqs_per_chunk = _CHUNK // S
    nvec = D // _L
    c0 = wid * n_chunks

    def fire(c, b):
        pltpu.sync_copy(ids_hbm.at[pl.ds(c * _SPC, _SPC)], idx_v.at[b])
        for k in range(_SPC):
            pltpu.async_copy(tok_hbm.at[idx_v.at[b, k]],
                             rows_v.at[b, pl.ds(k * _SUB, _SUB)], gsem[b])

    def drain_gather(b):
        pltpu.make_async_copy(tok_hbm.at[pl.ds(0, _CHUNK)],
                              rows_v.at[b], gsem[b]).wait()

    def fire_out(c, b):
        pltpu.async_copy(rows_v.at[b],
                         out_hbm.at[pl.ds(c * _CHUNK, _CHUNK)], osem[b])

    def drain_out(b):
        pltpu.make_async_copy(rows_v.at[b],
                              out_hbm.at[pl.ds(0, _CHUNK)], osem[b]).wait()

    def add_pos(b):
        def pos_body(p, carry):
            pv = [pos_v[p, pl.ds(j * _L, _L)] for j in range(nvec)]
            for s_i in range(seqs_per_chunk):
                r = s_i * S + p
                for j in range(nvec):
                    rows_v[b, r, pl.ds(j * _L, _L)] = (
                        rows_v[b, r, pl.ds(j * _L, _L)] + pv[j])
            return carry
        lax.fori_loop(0, S, pos_body, 0)

    fire(c0, 0)
    n_groups = n_chunks // _NBUF

    def group(g, carry):
        for b in range(_NBUF):
            c = c0 + g * _NBUF + b
            nb = (b + 1) % _NBUF
            if b == _NBUF - 1:
                drain_out(nb)
                @pl.when(g < n_groups - 1)
                def _():
                    fire(c + 1, nb)
            else:
                @pl.when(g >= 1)
                def _():
                    drain_out(nb)
                fire(c + 1, nb)
            drain_gather(b)
            add_pos(b)
            fire_out(c, b)
        return carry

    lax.fori_loop(0, n_groups, group, 0)
    for b in range(1, _NBUF):
        drain_out(b)


def kernel(input_ids, token_table, position_table):
    B, S = input_ids.shape
    V, D = token_table.shape
    N = B * S
    per_w = N // _NW
    assert N % (_NW * _CHUNK) == 0 and per_w % S == 0
    assert S % _CHUNK == 0 or _CHUNK % S == 0
    assert N % _SUB == 0 and D % _L == 0
    n_chunks = per_w // _CHUNK
    assert n_chunks % _NBUF == 0

    ids2 = input_ids.reshape(N // _SUB, _SUB).astype(jnp.int32)
    mesh = plsc.VectorSubcoreMesh(core_axis_name="c", subcore_axis_name="s",
                                  num_cores=_NC, num_subcores=_NS)
    k = pl.kernel(
        functools.partial(_emb_body, S, D, n_chunks),
        out_type=jax.ShapeDtypeStruct((N, D), jnp.float32),
        mesh=mesh,
        scratch_types=[
            pltpu.VMEM((_NBUF, _SPC, _SUB), jnp.int32),
            pltpu.VMEM((_NBUF, _CHUNK, D), jnp.float32),
            pltpu.VMEM((S, D), jnp.float32),
        ] + [pltpu.SemaphoreType.DMA] * (2 * _NBUF),
        compiler_params=pltpu.CompilerParams(use_tc_tiling_on_sc=False),
    )
    out = k(ids2, token_table, position_table.astype(jnp.float32))
    return out.reshape(B, S, D)
